# Initial kernel scaffold; baseline (speedup 1.0000x reference)
import jax
import jax.numpy as jnp
from jax.experimental import pallas as pl


def kernel(x, edge_index, W1, b1, W2, b2, Wc, bc):
    raise NotImplementedError("write your pallas kernel here")


# speedup vs baseline: 2.3197x; 2.3197x over previous
import functools

import jax
import jax.numpy as jnp
from jax import lax
from jax.experimental import pallas as pl
from jax.experimental.pallas import tpu as pltpu
from jax.experimental.pallas import tpu_sc as plsc

_NC = 2
_NS = 16
_K = 128
_BM = 256

_N = 10000
_NPAD = 10240
_ROWS_PER_SUB = _NPAD // _NS


def _sc_mesh():
    return plsc.VectorSubcoreMesh(
        core_axis_name="c", subcore_axis_name="s", num_cores=_NC,
        num_subcores=_NS)


def _sc_degrees(src_e, dst_e, onesA_hbm, onesB_hbm, z128_hbm, nb):
    @functools.partial(
        pl.kernel,
        out_type=jax.ShapeDtypeStruct((_NC, _NPAD, 128), jnp.float32),
        mesh=_sc_mesh(),
        scratch_types=[
            pltpu.VMEM((nb, _K), jnp.int32),
            pltpu.VMEM((nb, _K), jnp.int32),
            pltpu.VMEM((_K, 128), jnp.float32),
            pltpu.VMEM((_K, 128), jnp.float32),
            pltpu.VMEM_SHARED((_NPAD, 128), jnp.float32),
        ],
    )
    def k(src_hbm, dst_hbm, onesA_in, onesB_in, z_in, out_hbm,
          srcv, dstv, onesA, onesB, acc):
        cid = lax.axis_index("c")
        sid = lax.axis_index("s")
        wid = cid * _NS + sid
        pltpu.sync_copy(src_hbm.at[wid], srcv)
        pltpu.sync_copy(dst_hbm.at[wid], dstv)
        pltpu.sync_copy(onesA_in, onesA)
        pltpu.sync_copy(onesB_in, onesB)
        r0 = sid * _ROWS_PER_SUB
        def zero(i, c):
            pltpu.sync_copy(z_in, acc.at[pl.ds(r0 + i * _K, _K)])
            return c
        lax.fori_loop(0, _ROWS_PER_SUB // _K, zero, 0)
        plsc.subcore_barrier()
        def scat(j, c):
            pltpu.sync_copy(onesA, acc.at[srcv.at[j]], add=True)
            pltpu.sync_copy(onesB, acc.at[dstv.at[j]], add=True)
            return c
        lax.fori_loop(0, nb, scat, 0)
        plsc.subcore_barrier()
        def wout(i, c):
            r = r0 + i * _K
            pltpu.sync_copy(acc.at[pl.ds(r, _K)], onesA)
            pltpu.sync_copy(onesA, out_hbm.at[cid].at[pl.ds(r, _K)])
            return c
        lax.fori_loop(0, _ROWS_PER_SUB // _K, wout, 0)

    return k(src_e, dst_e, onesA_hbm, onesB_hbm, z128_hbm)


def _sc_segsum_slices(table, src_r, dst_r, z128_hbm, nsl, nb):
    nsl_sc = nsl // _NC

    @functools.partial(
        pl.kernel,
        out_type=jax.ShapeDtypeStruct((nsl, _NPAD, 128), jnp.float32),
        mesh=_sc_mesh(),
        scratch_types=[
            pltpu.VMEM((nb, _K), jnp.int32),
            pltpu.VMEM((nb, _K), jnp.int32),
            pltpu.VMEM((_K, 128), jnp.float32),
            pltpu.VMEM_SHARED((_NPAD, 128), jnp.float32),
            pltpu.SemaphoreType.DMA,
        ],
    )
    def k(tab_hbm, src_hbm, dst_hbm, z_in, out_hbm,
          srcv, dstv, rows, acc, sem):
        cid = lax.axis_index("c")
        sid = lax.axis_index("s")
        pltpu.sync_copy(src_hbm.at[sid], srcv)
        pltpu.sync_copy(dst_hbm.at[sid], dstv)
        r0 = sid * _ROWS_PER_SUB
        for t in range(nsl_sc):
            sl = cid * nsl_sc + t
            def zero(i, c):
                pltpu.sync_copy(z_in, acc.at[pl.ds(r0 + i * _K, _K)])
                return c
            lax.fori_loop(0, _ROWS_PER_SUB // _K, zero, 0)
            plsc.subcore_barrier()
            tab = tab_hbm.at[sl]
            def scat(j, c):
                pltpu.async_copy(tab.at[srcv.at[j]], rows, sem).wait()
                pltpu.sync_copy(rows, acc.at[dstv.at[j]], add=True)
                return c
            lax.fori_loop(0, nb, scat, 0)
            plsc.subcore_barrier()
            def wout(i, c):
                r = r0 + i * _K
                pltpu.sync_copy(acc.at[pl.ds(r, _K)], rows)
                pltpu.sync_copy(rows, out_hbm.at[sl].at[pl.ds(r, _K)])
                return c
            lax.fori_loop(0, _ROWS_PER_SUB // _K, wout, 0)
            if t + 1 < nsl_sc:
                plsc.subcore_barrier()

    return k(table, src_r, dst_r, z128_hbm)


def _sc_segsum_esplit(table, src_e, dst_e, z128_hbm, nb):
    @functools.partial(
        pl.kernel,
        out_type=jax.ShapeDtypeStruct((_NC, _NPAD, 128), jnp.float32),
        mesh=_sc_mesh(),
        scratch_types=[
            pltpu.VMEM((nb, _K), jnp.int32),
            pltpu.VMEM((nb, _K), jnp.int32),
            pltpu.VMEM((_K, 128), jnp.float32),
            pltpu.VMEM_SHARED((_NPAD, 128), jnp.float32),
            pltpu.SemaphoreType.DMA,
        ],
    )
    def k(tab_hbm, src_hbm, dst_hbm, z_in, out_hbm,
          srcv, dstv, rows, acc, sem):
        cid = lax.axis_index("c")
        sid = lax.axis_index("s")
        wid = cid * _NS + sid
        pltpu.sync_copy(src_hbm.at[wid], srcv)
        pltpu.sync_copy(dst_hbm.at[wid], dstv)
        r0 = sid * _ROWS_PER_SUB
        def zero(i, c):
            pltpu.sync_copy(z_in, acc.at[pl.ds(r0 + i * _K, _K)])
            return c
        lax.fori_loop(0, _ROWS_PER_SUB // _K, zero, 0)
        plsc.subcore_barrier()
        def scat(j, c):
            pltpu.async_copy(tab_hbm.at[srcv.at[j]], rows, sem).wait()
            pltpu.sync_copy(rows, acc.at[dstv.at[j]], add=True)
            return c
        lax.fori_loop(0, nb, scat, 0)
        plsc.subcore_barrier()
        def wout(i, c):
            r = r0 + i * _K
            pltpu.sync_copy(acc.at[pl.ds(r, _K)], rows)
            pltpu.sync_copy(rows, out_hbm.at[cid].at[pl.ds(r, _K)])
            return c
        lax.fori_loop(0, _ROWS_PER_SUB // _K, wout, 0)

    return k(table, src_e, dst_e, z128_hbm)


def _tc_norms_xs(xpad, degs):
    def body(x_ref, degs_ref, xs_ref, ns_ref, nd_ref):
        dsrc = degs_ref[0, :, 0] + degs_ref[1, :, 0]
        ddst = degs_ref[0, :, 1] + degs_ref[1, :, 1]
        ns = lax.rsqrt(jnp.maximum(dsrc, 1.0))
        nd = lax.rsqrt(jnp.maximum(ddst, 1.0))
        ns_ref[...] = ns[:, None]
        nd_ref[...] = nd[:, None]
        xs_ref[0] = x_ref[...] * ns[:, None]

    grid = (_NPAD // _BM, 2)
    return pl.pallas_call(
        body,
        grid=grid,
        in_specs=[
            pl.BlockSpec((_BM, 128), lambda i, j: (i, j)),
            pl.BlockSpec((_NC, _BM, 128), lambda i, j: (0, i, 0)),
        ],
        out_specs=[
            pl.BlockSpec((1, _BM, 128), lambda i, j: (j, i, 0)),
            pl.BlockSpec((_BM, 1), lambda i, j: (i, 0)),
            pl.BlockSpec((_BM, 1), lambda i, j: (i, 0)),
        ],
        out_shape=[
            jax.ShapeDtypeStruct((2, _NPAD, 128), jnp.float32),
            jax.ShapeDtypeStruct((_NPAD, 1), jnp.float32),
            jax.ShapeDtypeStruct((_NPAD, 1), jnp.float32),
        ],
    )(xpad, degs)


def _tc_conv(agg_t, nd, ns, W, b, nsl_in, nsl_out):
    K = nsl_in * 128
    b_r = b.reshape(nsl_out, 1, 128)

    def body(agg_ref, nd_ref, ns_ref, W_ref, b_ref, out_ref):
        z = jnp.zeros((_BM, 128), jnp.float32)
        for s in range(nsl_in):
            z = z + jnp.dot(agg_ref[s], W_ref[s * 128:(s + 1) * 128, :],
                            preferred_element_type=jnp.float32)
        z = z * nd_ref[...] + b_ref[0]
        out_ref[0] = jnp.maximum(z, 0.0) * ns_ref[...]

    grid = (_NPAD // _BM, nsl_out)
    return pl.pallas_call(
        body,
        grid=grid,
        in_specs=[
            pl.BlockSpec((nsl_in, _BM, 128), lambda i, j: (0, i, 0)),
            pl.BlockSpec((_BM, 1), lambda i, j: (i, 0)),
            pl.BlockSpec((_BM, 1), lambda i, j: (i, 0)),
            pl.BlockSpec((K, 128), lambda i, j: (0, j)),
            pl.BlockSpec((1, 1, 128), lambda i, j: (j, 0, 0)),
        ],
        out_specs=pl.BlockSpec((1, _BM, 128), lambda i, j: (j, i, 0)),
        out_shape=jax.ShapeDtypeStruct((nsl_out, _NPAD, 128), jnp.float32),
    )(agg_t, nd, ns, W, b_r)


def _tc_proj(h2s_t, Wcp, nsl_in):
    def body(h_ref, W_ref, out_ref):
        z = jnp.zeros((_BM, 128), jnp.float32)
        for s in range(nsl_in):
            z = z + jnp.dot(h_ref[s], W_ref[s * 128:(s + 1) * 128, :],
                            preferred_element_type=jnp.float32)
        out_ref[...] = z

    grid = (_NPAD // _BM,)
    return pl.pallas_call(
        body,
        grid=grid,
        in_specs=[
            pl.BlockSpec((nsl_in, _BM, 128), lambda i: (0, i, 0)),
            pl.BlockSpec((nsl_in * 128, 128), lambda i: (0, 0)),
        ],
        out_specs=pl.BlockSpec((_BM, 128), lambda i: (i, 0)),
        out_shape=jax.ShapeDtypeStruct((_NPAD, 128), jnp.float32),
    )(h2s_t, Wcp)


def _tc_softmax(agg3p, nd, bc):
    bc_r = bc.reshape(1, 64)

    def body(p_ref, nd_ref, bc_ref, out_ref):
        z = (p_ref[0, :, :64] + p_ref[1, :, :64]) * nd_ref[...] + bc_ref[...]
        m = jnp.max(z, axis=1, keepdims=True)
        e = jnp.exp(z - m)
        out_ref[...] = e / jnp.sum(e, axis=1, keepdims=True)

    grid = (_NPAD // _BM,)
    return pl.pallas_call(
        body,
        grid=grid,
        in_specs=[
            pl.BlockSpec((_NC, _BM, 128), lambda i: (0, i, 0)),
            pl.BlockSpec((_BM, 1), lambda i: (i, 0)),
            pl.BlockSpec((1, 64), lambda i: (0, 0)),
        ],
        out_specs=pl.BlockSpec((_BM, 64), lambda i: (i, 0)),
        out_shape=jax.ShapeDtypeStruct((_NPAD, 64), jnp.float32),
    )(agg3p, nd, bc_r)


def kernel(x, edge_index, W1, b1, W2, b2, Wc, bc):
    N = x.shape[0]
    E = edge_index.shape[1]
    d_in = x.shape[1]

    epad = -(-E // (_NC * _NS * _K)) * (_NC * _NS * _K)
    src = edge_index[0]
    dst = edge_index[1]
    pad = epad - E
    srcp = jnp.concatenate([src, jnp.full((pad,), N, jnp.int32)])
    dstp = jnp.concatenate([dst, jnp.full((pad,), N, jnp.int32)])
    src16 = srcp.reshape(_NS, epad // (_NS * _K), _K)
    dst16 = dstp.reshape(_NS, epad // (_NS * _K), _K)
    src32 = srcp.reshape(_NC * _NS, epad // (_NC * _NS * _K), _K)
    dst32 = dstp.reshape(_NC * _NS, epad // (_NC * _NS * _K), _K)
    nb16 = epad // (_NS * _K)
    nb32 = epad // (_NC * _NS * _K)

    xpad = jnp.pad(x, ((0, _NPAD - N), (0, 0)))
    z128 = jnp.zeros((_K, 128), jnp.float32)
    onesA = jnp.zeros((_K, 128), jnp.float32).at[:, 0].set(1.0)
    onesB = jnp.zeros((_K, 128), jnp.float32).at[:, 1].set(1.0)
    Wcp = jnp.pad(Wc, ((0, 0), (0, 128 - Wc.shape[1])))

    degs = _sc_degrees(src32, dst32, onesA, onesB, z128, nb32)
    xs_t, ns, nd = _tc_norms_xs(xpad, degs)
    agg1 = _sc_segsum_slices(xs_t, src16, dst16, z128, d_in // 128, nb16)
    hs1 = _tc_conv(agg1, nd, ns, W1, b1, d_in // 128, W1.shape[1] // 128)
    agg2 = _sc_segsum_slices(hs1, src16, dst16, z128, W1.shape[1] // 128, nb16)
    h2s = _tc_conv(agg2, nd, ns, W2, b2, W2.shape[0] // 128, W2.shape[1] // 128)
    t3 = _tc_proj(h2s, Wcp, W2.shape[1] // 128)
    agg3p = _sc_segsum_esplit(t3, src32, dst32, z128, nb32)
    probs = _tc_softmax(agg3p, nd, bc)
    return probs[:N]


# speedup vs baseline: 2.6329x; 1.1350x over previous
import functools

import jax
import jax.numpy as jnp
from jax import lax
from jax.experimental import pallas as pl
from jax.experimental.pallas import tpu as pltpu
from jax.experimental.pallas import tpu_sc as plsc

_NC = 2
_NS = 16
_K = 128
_BK = 128
_CH = 40
_BM = 256

_N = 10000
_NPAD = 10240
_ROWS_PER_SUB = _NPAD // _NS


def _sc_mesh():
    return plsc.VectorSubcoreMesh(
        core_axis_name="c", subcore_axis_name="s", num_cores=_NC,
        num_subcores=_NS)


def _sc_degrees(src_e, dst_e, onesA_hbm, onesB_hbm, z128_hbm, nb):
    @functools.partial(
        pl.kernel,
        out_type=jax.ShapeDtypeStruct((_NC, _NPAD, 128), jnp.float32),
        mesh=_sc_mesh(),
        scratch_types=[
            pltpu.VMEM((nb, _BK), jnp.int32),
            pltpu.VMEM((nb, _BK), jnp.int32),
            pltpu.VMEM((_BK, 128), jnp.float32),
            pltpu.VMEM((_BK, 128), jnp.float32),
            pltpu.VMEM_SHARED((_NPAD, 128), jnp.float32),
        ],
    )
    def k(src_hbm, dst_hbm, onesA_in, onesB_in, z_in, out_hbm,
          srcv, dstv, onesA, onesB, acc):
        cid = lax.axis_index("c")
        sid = lax.axis_index("s")
        wid = cid * _NS + sid
        pltpu.sync_copy(src_hbm.at[wid], srcv)
        pltpu.sync_copy(dst_hbm.at[wid], dstv)
        pltpu.sync_copy(onesA_in, onesA)
        pltpu.sync_copy(onesB_in, onesB)
        r0 = sid * _ROWS_PER_SUB
        def zero(i, c):
            pltpu.sync_copy(z_in, acc.at[pl.ds(r0 + i * _K, _K)])
            return c
        lax.fori_loop(0, _ROWS_PER_SUB // _K, zero, 0)
        plsc.subcore_barrier()
        def scat(j, c):
            pltpu.sync_copy(onesA, acc.at[srcv.at[j]], add=True)
            pltpu.sync_copy(onesB, acc.at[dstv.at[j]], add=True)
            return c
        lax.fori_loop(0, nb, scat, 0)
        plsc.subcore_barrier()
        def wout(i, c):
            r = r0 + i * _BK
            pltpu.sync_copy(acc.at[pl.ds(r, _BK)], onesA)
            pltpu.sync_copy(onesA, out_hbm.at[cid].at[pl.ds(r, _BK)])
            return c
        lax.fori_loop(0, _ROWS_PER_SUB // _BK, wout, 0)

    return k(src_e, dst_e, onesA_hbm, onesB_hbm, z128_hbm)


def _sc_segsum_slices(table, src_r, dst_r, z128_hbm, nsl, nb):
    nsl_sc = nsl // _NC
    nch = nb // _CH

    @functools.partial(
        pl.kernel,
        out_type=jax.ShapeDtypeStruct((nsl, _NPAD, 128), jnp.float32),
        mesh=_sc_mesh(),
        scratch_types=[
            pltpu.VMEM((_CH, _BK), jnp.int32),
            pltpu.VMEM((_CH, _BK), jnp.int32),
            pltpu.VMEM((_BK, 128), jnp.float32),
            pltpu.VMEM((_BK, 128), jnp.float32),
            pltpu.VMEM_SHARED((_NPAD, 128), jnp.float32),
            pltpu.SemaphoreType.DMA,
            pltpu.SemaphoreType.DMA,
        ],
    )
    def k(tab_hbm, src_hbm, dst_hbm, z_in, out_hbm,
          srcv, dstv, buf0, buf1, acc, sem0, sem1):
        cid = lax.axis_index("c")
        sid = lax.axis_index("s")
        bufs = (buf0, buf1)
        sems = (sem0, sem1)
        r0 = sid * _ROWS_PER_SUB
        for t in range(nsl_sc):
            sl = cid * nsl_sc + t
            def zero(i, c):
                pltpu.sync_copy(z_in, acc.at[pl.ds(r0 + i * _K, _K)])
                return c
            lax.fori_loop(0, _ROWS_PER_SUB // _K, zero, 0)
            plsc.subcore_barrier()
            tab = tab_hbm.at[sl]
            for h in range(nch):
                pltpu.sync_copy(src_hbm.at[sid].at[pl.ds(h * _CH, _CH)], srcv)
                pltpu.sync_copy(dst_hbm.at[sid].at[pl.ds(h * _CH, _CH)], dstv)
                pltpu.async_copy(tab.at[srcv.at[0]], buf0, sem0)
                pltpu.async_copy(tab.at[srcv.at[1]], buf1, sem1)
                def outer(i0, c):
                    for b in range(2):
                        j = 2 * i0 + b
                        pltpu.make_async_copy(
                            tab.at[srcv.at[j]], bufs[b], sems[b]).wait()
                        pltpu.sync_copy(bufs[b], acc.at[dstv.at[j]], add=True)
                        @pl.when(j + 2 < _CH)
                        def _():
                            pltpu.async_copy(
                                tab.at[srcv.at[j + 2]], bufs[b], sems[b])
                    return c
                lax.fori_loop(0, _CH // 2, outer, 0)
            plsc.subcore_barrier()
            def wout(i, c):
                r = r0 + i * _BK
                pltpu.sync_copy(acc.at[pl.ds(r, _BK)], buf0)
                pltpu.sync_copy(buf0, out_hbm.at[sl].at[pl.ds(r, _BK)])
                return c
            lax.fori_loop(0, _ROWS_PER_SUB // _BK, wout, 0)
            if t + 1 < nsl_sc:
                plsc.subcore_barrier()

    return k(table, src_r, dst_r, z128_hbm)


def _sc_segsum_esplit(table, src_e, dst_e, z128_hbm, nb):
    @functools.partial(
        pl.kernel,
        out_type=jax.ShapeDtypeStruct((_NC, _NPAD, 128), jnp.float32),
        mesh=_sc_mesh(),
        scratch_types=[
            pltpu.VMEM((nb, _BK), jnp.int32),
            pltpu.VMEM((nb, _BK), jnp.int32),
            pltpu.VMEM((_BK, 128), jnp.float32),
            pltpu.VMEM((_BK, 128), jnp.float32),
            pltpu.VMEM_SHARED((_NPAD, 128), jnp.float32),
            pltpu.SemaphoreType.DMA,
            pltpu.SemaphoreType.DMA,
        ],
    )
    def k(tab_hbm, src_hbm, dst_hbm, z_in, out_hbm,
          srcv, dstv, buf0, buf1, acc, sem0, sem1):
        cid = lax.axis_index("c")
        sid = lax.axis_index("s")
        wid = cid * _NS + sid
        pltpu.sync_copy(src_hbm.at[wid], srcv)
        pltpu.sync_copy(dst_hbm.at[wid], dstv)
        bufs = (buf0, buf1)
        sems = (sem0, sem1)
        r0 = sid * _ROWS_PER_SUB
        def zero(i, c):
            pltpu.sync_copy(z_in, acc.at[pl.ds(r0 + i * _K, _K)])
            return c
        lax.fori_loop(0, _ROWS_PER_SUB // _K, zero, 0)
        plsc.subcore_barrier()
        pltpu.async_copy(tab_hbm.at[srcv.at[0]], buf0, sem0)
        pltpu.async_copy(tab_hbm.at[srcv.at[1]], buf1, sem1)
        def outer(i0, c):
            for b in range(2):
                j = 2 * i0 + b
                pltpu.make_async_copy(
                    tab_hbm.at[srcv.at[j]], bufs[b], sems[b]).wait()
                pltpu.sync_copy(bufs[b], acc.at[dstv.at[j]], add=True)
                @pl.when(j + 2 < nb)
                def _():
                    pltpu.async_copy(
                        tab_hbm.at[srcv.at[j + 2]], bufs[b], sems[b])
            return c
        lax.fori_loop(0, nb // 2, outer, 0)
        plsc.subcore_barrier()
        def wout(i, c):
            r = r0 + i * _BK
            pltpu.sync_copy(acc.at[pl.ds(r, _BK)], buf0)
            pltpu.sync_copy(buf0, out_hbm.at[cid].at[pl.ds(r, _BK)])
            return c
        lax.fori_loop(0, _ROWS_PER_SUB // _BK, wout, 0)

    return k(table, src_e, dst_e, z128_hbm)


def _tc_norms_xs(xpad, degs):
    def body(x_ref, degs_ref, xs_ref, ns_ref, nd_ref):
        dsrc = degs_ref[0, :, 0] + degs_ref[1, :, 0]
        ddst = degs_ref[0, :, 1] + degs_ref[1, :, 1]
        ns = lax.rsqrt(jnp.maximum(dsrc, 1.0))
        nd = lax.rsqrt(jnp.maximum(ddst, 1.0))
        ns_ref[...] = ns[:, None]
        nd_ref[...] = nd[:, None]
        xs_ref[0] = x_ref[...] * ns[:, None]

    grid = (_NPAD // _BM, 2)
    return pl.pallas_call(
        body,
        grid=grid,
        in_specs=[
            pl.BlockSpec((_BM, 128), lambda i, j: (i, j)),
            pl.BlockSpec((_NC, _BM, 128), lambda i, j: (0, i, 0)),
        ],
        out_specs=[
            pl.BlockSpec((1, _BM, 128), lambda i, j: (j, i, 0)),
            pl.BlockSpec((_BM, 1), lambda i, j: (i, 0)),
            pl.BlockSpec((_BM, 1), lambda i, j: (i, 0)),
        ],
        out_shape=[
            jax.ShapeDtypeStruct((2, _NPAD, 128), jnp.float32),
            jax.ShapeDtypeStruct((_NPAD, 1), jnp.float32),
            jax.ShapeDtypeStruct((_NPAD, 1), jnp.float32),
        ],
    )(xpad, degs)


def _tc_conv(agg_t, nd, ns, W, b, nsl_in, nsl_out):
    K = nsl_in * 128
    b_r = b.reshape(nsl_out, 1, 128)

    def body(agg_ref, nd_ref, ns_ref, W_ref, b_ref, out_ref):
        z = jnp.zeros((_BM, 128), jnp.float32)
        for s in range(nsl_in):
            z = z + jnp.dot(agg_ref[s], W_ref[s * 128:(s + 1) * 128, :],
                            preferred_element_type=jnp.float32)
        z = z * nd_ref[...] + b_ref[0]
        out_ref[0] = jnp.maximum(z, 0.0) * ns_ref[...]

    grid = (_NPAD // _BM, nsl_out)
    return pl.pallas_call(
        body,
        grid=grid,
        in_specs=[
            pl.BlockSpec((nsl_in, _BM, 128), lambda i, j: (0, i, 0)),
            pl.BlockSpec((_BM, 1), lambda i, j: (i, 0)),
            pl.BlockSpec((_BM, 1), lambda i, j: (i, 0)),
            pl.BlockSpec((K, 128), lambda i, j: (0, j)),
            pl.BlockSpec((1, 1, 128), lambda i, j: (j, 0, 0)),
        ],
        out_specs=pl.BlockSpec((1, _BM, 128), lambda i, j: (j, i, 0)),
        out_shape=jax.ShapeDtypeStruct((nsl_out, _NPAD, 128), jnp.float32),
    )(agg_t, nd, ns, W, b_r)


def _tc_proj(h2s_t, Wcp, nsl_in):
    def body(h_ref, W_ref, out_ref):
        z = jnp.zeros((_BM, 128), jnp.float32)
        for s in range(nsl_in):
            z = z + jnp.dot(h_ref[s], W_ref[s * 128:(s + 1) * 128, :],
                            preferred_element_type=jnp.float32)
        out_ref[...] = z

    grid = (_NPAD // _BM,)
    return pl.pallas_call(
        body,
        grid=grid,
        in_specs=[
            pl.BlockSpec((nsl_in, _BM, 128), lambda i: (0, i, 0)),
            pl.BlockSpec((nsl_in * 128, 128), lambda i: (0, 0)),
        ],
        out_specs=pl.BlockSpec((_BM, 128), lambda i: (i, 0)),
        out_shape=jax.ShapeDtypeStruct((_NPAD, 128), jnp.float32),
    )(h2s_t, Wcp)


def _tc_softmax(agg3p, nd, bc):
    bc_r = bc.reshape(1, 64)

    def body(p_ref, nd_ref, bc_ref, out_ref):
        z = (p_ref[0, :, :64] + p_ref[1, :, :64]) * nd_ref[...] + bc_ref[...]
        m = jnp.max(z, axis=1, keepdims=True)
        e = jnp.exp(z - m)
        out_ref[...] = e / jnp.sum(e, axis=1, keepdims=True)

    grid = (_NPAD // _BM,)
    return pl.pallas_call(
        body,
        grid=grid,
        in_specs=[
            pl.BlockSpec((_NC, _BM, 128), lambda i: (0, i, 0)),
            pl.BlockSpec((_BM, 1), lambda i: (i, 0)),
            pl.BlockSpec((1, 64), lambda i: (0, 0)),
        ],
        out_specs=pl.BlockSpec((_BM, 64), lambda i: (i, 0)),
        out_shape=jax.ShapeDtypeStruct((_NPAD, 64), jnp.float32),
    )(agg3p, nd, bc_r)


def kernel(x, edge_index, W1, b1, W2, b2, Wc, bc):
    N = x.shape[0]
    E = edge_index.shape[1]
    d_in = x.shape[1]

    epad = -(-E // (_NC * _NS * _BK)) * (_NC * _NS * _BK)
    src = edge_index[0]
    dst = edge_index[1]
    pad = epad - E
    srcp = jnp.concatenate([src, jnp.full((pad,), N, jnp.int32)])
    dstp = jnp.concatenate([dst, jnp.full((pad,), N, jnp.int32)])
    src16 = srcp.reshape(_NS, epad // (_NS * _BK), _BK)
    dst16 = dstp.reshape(_NS, epad // (_NS * _BK), _BK)
    src32 = srcp.reshape(_NC * _NS, epad // (_NC * _NS * _BK), _BK)
    dst32 = dstp.reshape(_NC * _NS, epad // (_NC * _NS * _BK), _BK)
    nb16 = epad // (_NS * _BK)
    nb32 = epad // (_NC * _NS * _BK)

    xpad = jnp.pad(x, ((0, _NPAD - N), (0, 0)))
    z128 = jnp.zeros((_K, 128), jnp.float32)
    onesA = jnp.zeros((_BK, 128), jnp.float32).at[:, 0].set(1.0)
    onesB = jnp.zeros((_BK, 128), jnp.float32).at[:, 1].set(1.0)
    Wcp = jnp.pad(Wc, ((0, 0), (0, 128 - Wc.shape[1])))

    degs = _sc_degrees(src32, dst32, onesA, onesB, z128, nb32)
    xs_t, ns, nd = _tc_norms_xs(xpad, degs)
    agg1 = _sc_segsum_slices(xs_t, src16, dst16, z128, d_in // 128, nb16)
    hs1 = _tc_conv(agg1, nd, ns, W1, b1, d_in // 128, W1.shape[1] // 128)
    agg2 = _sc_segsum_slices(hs1, src16, dst16, z128, W1.shape[1] // 128, nb16)
    h2s = _tc_conv(agg2, nd, ns, W2, b2, W2.shape[0] // 128, W2.shape[1] // 128)
    t3 = _tc_proj(h2s, Wcp, W2.shape[1] // 128)
    agg3p = _sc_segsum_esplit(t3, src32, dst32, z128, nb32)
    probs = _tc_softmax(agg3p, nd, bc)
    return probs[:N]


# speedup vs baseline: 2.8593x; 1.0860x over previous
import functools

import jax
import jax.numpy as jnp
from jax import lax
from jax.experimental import pallas as pl
from jax.experimental.pallas import tpu as pltpu
from jax.experimental.pallas import tpu_sc as plsc

_NC = 2
_NS = 16
_K = 128
_BK = 128
_CH = 40
_BM = 256

_N = 10000
_NPAD = 10240
_ROWS_PER_SUB = _NPAD // _NS


def _sc_mesh():
    return plsc.VectorSubcoreMesh(
        core_axis_name="c", subcore_axis_name="s", num_cores=_NC,
        num_subcores=_NS)


def _sc_degrees(src_e, dst_e, onesA_hbm, onesB_hbm, z128_hbm, nb):
    @functools.partial(
        pl.kernel,
        out_type=jax.ShapeDtypeStruct((_NC, _NPAD, 128), jnp.float32),
        mesh=_sc_mesh(),
        scratch_types=[
            pltpu.VMEM((nb, _BK), jnp.int32),
            pltpu.VMEM((nb, _BK), jnp.int32),
            pltpu.VMEM((_BK, 128), jnp.float32),
            pltpu.VMEM((_BK, 128), jnp.float32),
            pltpu.VMEM_SHARED((_NPAD, 128), jnp.float32),
        ],
    )
    def k(src_hbm, dst_hbm, onesA_in, onesB_in, z_in, out_hbm,
          srcv, dstv, onesA, onesB, acc):
        cid = lax.axis_index("c")
        sid = lax.axis_index("s")
        wid = cid * _NS + sid
        pltpu.sync_copy(src_hbm.at[wid], srcv)
        pltpu.sync_copy(dst_hbm.at[wid], dstv)
        pltpu.sync_copy(onesA_in, onesA)
        pltpu.sync_copy(onesB_in, onesB)
        r0 = sid * _ROWS_PER_SUB
        def zero(i, c):
            pltpu.sync_copy(z_in, acc.at[pl.ds(r0 + i * _K, _K)])
            return c
        lax.fori_loop(0, _ROWS_PER_SUB // _K, zero, 0)
        plsc.subcore_barrier()
        def scat(j, c):
            pltpu.sync_copy(onesA, acc.at[srcv.at[j]], add=True)
            pltpu.sync_copy(onesB, acc.at[dstv.at[j]], add=True)
            return c
        lax.fori_loop(0, nb, scat, 0)
        plsc.subcore_barrier()
        def wout(i, c):
            r = r0 + i * _BK
            pltpu.sync_copy(acc.at[pl.ds(r, _BK)], onesA)
            pltpu.sync_copy(onesA, out_hbm.at[cid].at[pl.ds(r, _BK)])
            return c
        lax.fori_loop(0, _ROWS_PER_SUB // _BK, wout, 0)

    return k(src_e, dst_e, onesA_hbm, onesB_hbm, z128_hbm)


def _sc_segsum_slices(table, src_r, dst_r, z128_hbm, nsl, nb):
    nsl_sc = nsl // _NC
    nch = nb // _CH

    @functools.partial(
        pl.kernel,
        out_type=jax.ShapeDtypeStruct((nsl, _NPAD, 128), jnp.float32),
        mesh=_sc_mesh(),
        scratch_types=[
            pltpu.VMEM((_CH, _BK), jnp.int32),
            pltpu.VMEM((_CH, _BK), jnp.int32),
            pltpu.VMEM((_BK, 128), jnp.float32),
            pltpu.VMEM((_BK, 128), jnp.float32),
            pltpu.VMEM_SHARED((_NPAD, 128), jnp.float32),
            pltpu.SemaphoreType.DMA,
            pltpu.SemaphoreType.DMA,
        ],
    )
    def k(tab_hbm, src_hbm, dst_hbm, z_in, out_hbm,
          srcv, dstv, buf0, buf1, acc, sem0, sem1):
        cid = lax.axis_index("c")
        sid = lax.axis_index("s")
        bufs = (buf0, buf1)
        sems = (sem0, sem1)
        r0 = sid * _ROWS_PER_SUB
        for t in range(nsl_sc):
            sl = cid * nsl_sc + t
            def zero(i, c):
                pltpu.sync_copy(z_in, acc.at[pl.ds(r0 + i * _K, _K)])
                return c
            lax.fori_loop(0, _ROWS_PER_SUB // _K, zero, 0)
            plsc.subcore_barrier()
            tab = tab_hbm.at[sl]
            for h in range(nch):
                pltpu.sync_copy(src_hbm.at[sid].at[pl.ds(h * _CH, _CH)], srcv)
                pltpu.sync_copy(dst_hbm.at[sid].at[pl.ds(h * _CH, _CH)], dstv)
                pltpu.async_copy(tab.at[srcv.at[0]], buf0, sem0)
                pltpu.async_copy(tab.at[srcv.at[1]], buf1, sem1)
                def outer(i0, c):
                    for b in range(2):
                        j = 2 * i0 + b
                        pltpu.make_async_copy(
                            tab.at[srcv.at[j]], bufs[b], sems[b]).wait()
                        pltpu.sync_copy(bufs[b], acc.at[dstv.at[j]], add=True)
                        @pl.when(j + 2 < _CH)
                        def _():
                            pltpu.async_copy(
                                tab.at[srcv.at[j + 2]], bufs[b], sems[b])
                    return c
                lax.fori_loop(0, _CH // 2, outer, 0)
            plsc.subcore_barrier()
            def wout(i, c):
                r = r0 + i * _BK
                pltpu.sync_copy(acc.at[pl.ds(r, _BK)], buf0)
                pltpu.sync_copy(buf0, out_hbm.at[sl].at[pl.ds(r, _BK)])
                return c
            lax.fori_loop(0, _ROWS_PER_SUB // _BK, wout, 0)
            if t + 1 < nsl_sc:
                plsc.subcore_barrier()

    return k(table, src_r, dst_r, z128_hbm)


def _sc_segsum_esplit(table, src_e, dst_e, z128_hbm, nb):
    @functools.partial(
        pl.kernel,
        out_type=jax.ShapeDtypeStruct((_NC, _NPAD, 128), jnp.float32),
        mesh=_sc_mesh(),
        scratch_types=[
            pltpu.VMEM((nb, _BK), jnp.int32),
            pltpu.VMEM((nb, _BK), jnp.int32),
            pltpu.VMEM((_BK, 128), jnp.float32),
            pltpu.VMEM((_BK, 128), jnp.float32),
            pltpu.VMEM_SHARED((_NPAD, 128), jnp.float32),
            pltpu.SemaphoreType.DMA,
            pltpu.SemaphoreType.DMA,
        ],
    )
    def k(tab_hbm, src_hbm, dst_hbm, z_in, out_hbm,
          srcv, dstv, buf0, buf1, acc, sem0, sem1):
        cid = lax.axis_index("c")
        sid = lax.axis_index("s")
        wid = cid * _NS + sid
        pltpu.sync_copy(src_hbm.at[wid], srcv)
        pltpu.sync_copy(dst_hbm.at[wid], dstv)
        bufs = (buf0, buf1)
        sems = (sem0, sem1)
        r0 = sid * _ROWS_PER_SUB
        def zero(i, c):
            pltpu.sync_copy(z_in, acc.at[pl.ds(r0 + i * _K, _K)])
            return c
        lax.fori_loop(0, _ROWS_PER_SUB // _K, zero, 0)
        plsc.subcore_barrier()
        pltpu.async_copy(tab_hbm.at[srcv.at[0]], buf0, sem0)
        pltpu.async_copy(tab_hbm.at[srcv.at[1]], buf1, sem1)
        def outer(i0, c):
            for b in range(2):
                j = 2 * i0 + b
                pltpu.make_async_copy(
                    tab_hbm.at[srcv.at[j]], bufs[b], sems[b]).wait()
                pltpu.sync_copy(bufs[b], acc.at[dstv.at[j]], add=True)
                @pl.when(j + 2 < nb)
                def _():
                    pltpu.async_copy(
                        tab_hbm.at[srcv.at[j + 2]], bufs[b], sems[b])
            return c
        lax.fori_loop(0, nb // 2, outer, 0)
        plsc.subcore_barrier()
        def wout(i, c):
            r = r0 + i * _BK
            pltpu.sync_copy(acc.at[pl.ds(r, _BK)], buf0)
            pltpu.sync_copy(buf0, out_hbm.at[cid].at[pl.ds(r, _BK)])
            return c
        lax.fori_loop(0, _ROWS_PER_SUB // _BK, wout, 0)

    return k(table, src_e, dst_e, z128_hbm)


def _tc_norms_xs(xpad, degs):
    def body(x_ref, degs_ref, xs_ref, ns_ref, nd_ref):
        dsrc = degs_ref[0, :, 0] + degs_ref[1, :, 0]
        ddst = degs_ref[0, :, 1] + degs_ref[1, :, 1]
        ns = lax.rsqrt(jnp.maximum(dsrc, 1.0))
        nd = lax.rsqrt(jnp.maximum(ddst, 1.0))
        ns_ref[...] = ns[:, None]
        nd_ref[...] = nd[:, None]
        xs_ref[0] = x_ref[...] * ns[:, None]

    grid = (_NPAD // _BM, 2)
    return pl.pallas_call(
        body,
        grid=grid,
        in_specs=[
            pl.BlockSpec((_BM, 128), lambda i, j: (i, j)),
            pl.BlockSpec((_NC, _BM, 128), lambda i, j: (0, i, 0)),
        ],
        out_specs=[
            pl.BlockSpec((1, _BM, 128), lambda i, j: (j, i, 0)),
            pl.BlockSpec((_BM, 1), lambda i, j: (i, 0)),
            pl.BlockSpec((_BM, 1), lambda i, j: (i, 0)),
        ],
        out_shape=[
            jax.ShapeDtypeStruct((2, _NPAD, 128), jnp.float32),
            jax.ShapeDtypeStruct((_NPAD, 1), jnp.float32),
            jax.ShapeDtypeStruct((_NPAD, 1), jnp.float32),
        ],
    )(xpad, degs)


def _tc_conv(agg_t, nd, ns, W, b, nsl_in, nsl_out):
    K = nsl_in * 128
    b_r = b.reshape(nsl_out, 1, 128)

    def body(agg_ref, nd_ref, ns_ref, W_ref, b_ref, out_ref):
        z = jnp.zeros((_BM, 128), jnp.float32)
        for s in range(nsl_in):
            z = z + jnp.dot(agg_ref[s], W_ref[s * 128:(s + 1) * 128, :],
                            preferred_element_type=jnp.float32)
        z = z * nd_ref[...] + b_ref[0]
        out_ref[0] = jnp.maximum(z, 0.0) * ns_ref[...]

    grid = (_NPAD // _BM, nsl_out)
    return pl.pallas_call(
        body,
        grid=grid,
        in_specs=[
            pl.BlockSpec((nsl_in, _BM, 128), lambda i, j: (0, i, 0)),
            pl.BlockSpec((_BM, 1), lambda i, j: (i, 0)),
            pl.BlockSpec((_BM, 1), lambda i, j: (i, 0)),
            pl.BlockSpec((K, 128), lambda i, j: (0, j)),
            pl.BlockSpec((1, 1, 128), lambda i, j: (j, 0, 0)),
        ],
        out_specs=pl.BlockSpec((1, _BM, 128), lambda i, j: (j, i, 0)),
        out_shape=jax.ShapeDtypeStruct((nsl_out, _NPAD, 128), jnp.float32),
    )(agg_t, nd, ns, W, b_r)


def _tc_conv_proj(agg_t, nd, ns, W2, b2, Wcp, nsl_in):
    K = nsl_in * 128
    b_r = b2.reshape(1, K)

    def body(agg_ref, nd_ref, ns_ref, W_ref, b_ref, Wc_ref, out_ref):
        z = jnp.zeros((_BM, K), jnp.float32)
        for s in range(nsl_in):
            z = z + jnp.dot(agg_ref[s], W_ref[s * 128:(s + 1) * 128, :],
                            preferred_element_type=jnp.float32)
        z = jnp.maximum(z * nd_ref[...] + b_ref[...], 0.0) * ns_ref[...]
        out_ref[...] = jnp.dot(z, Wc_ref[...],
                               preferred_element_type=jnp.float32)

    grid = (_NPAD // _BM,)
    return pl.pallas_call(
        body,
        grid=grid,
        in_specs=[
            pl.BlockSpec((nsl_in, _BM, 128), lambda i: (0, i, 0)),
            pl.BlockSpec((_BM, 1), lambda i: (i, 0)),
            pl.BlockSpec((_BM, 1), lambda i: (i, 0)),
            pl.BlockSpec((K, K), lambda i: (0, 0)),
            pl.BlockSpec((1, K), lambda i: (0, 0)),
            pl.BlockSpec((K, 128), lambda i: (0, 0)),
        ],
        out_specs=pl.BlockSpec((_BM, 128), lambda i: (i, 0)),
        out_shape=jax.ShapeDtypeStruct((_NPAD, 128), jnp.float32),
    )(agg_t, nd, ns, W2, b_r, Wcp)


def _tc_softmax(agg3p, nd, bc):
    bc_r = bc.reshape(1, 64)

    def body(p_ref, nd_ref, bc_ref, out_ref):
        z = (p_ref[0, :, :64] + p_ref[1, :, :64]) * nd_ref[...] + bc_ref[...]
        m = jnp.max(z, axis=1, keepdims=True)
        e = jnp.exp(z - m)
        out_ref[...] = e / jnp.sum(e, axis=1, keepdims=True)

    grid = (_NPAD // _BM,)
    return pl.pallas_call(
        body,
        grid=grid,
        in_specs=[
            pl.BlockSpec((_NC, _BM, 128), lambda i: (0, i, 0)),
            pl.BlockSpec((_BM, 1), lambda i: (i, 0)),
            pl.BlockSpec((1, 64), lambda i: (0, 0)),
        ],
        out_specs=pl.BlockSpec((_BM, 64), lambda i: (i, 0)),
        out_shape=jax.ShapeDtypeStruct((_NPAD, 64), jnp.float32),
    )(agg3p, nd, bc_r)


def kernel(x, edge_index, W1, b1, W2, b2, Wc, bc):
    N = x.shape[0]
    E = edge_index.shape[1]
    d_in = x.shape[1]

    epad = -(-E // (_NC * _NS * _BK)) * (_NC * _NS * _BK)
    src = edge_index[0]
    dst = edge_index[1]
    pad = epad - E
    srcp = jnp.concatenate([src, jnp.full((pad,), N, jnp.int32)])
    dstp = jnp.concatenate([dst, jnp.full((pad,), N, jnp.int32)])
    src16 = srcp.reshape(_NS, epad // (_NS * _BK), _BK)
    dst16 = dstp.reshape(_NS, epad // (_NS * _BK), _BK)
    src32 = srcp.reshape(_NC * _NS, epad // (_NC * _NS * _BK), _BK)
    dst32 = dstp.reshape(_NC * _NS, epad // (_NC * _NS * _BK), _BK)
    nb16 = epad // (_NS * _BK)
    nb32 = epad // (_NC * _NS * _BK)

    xpad = jnp.pad(x, ((0, _NPAD - N), (0, 0)))
    z128 = jnp.zeros((_K, 128), jnp.float32)
    onesA = jnp.zeros((_BK, 128), jnp.float32).at[:, 0].set(1.0)
    onesB = jnp.zeros((_BK, 128), jnp.float32).at[:, 1].set(1.0)
    Wcp = jnp.pad(Wc, ((0, 0), (0, 128 - Wc.shape[1])))

    degs = _sc_degrees(src32, dst32, onesA, onesB, z128, nb32)
    xs_t, ns, nd = _tc_norms_xs(xpad, degs)
    agg1 = _sc_segsum_slices(xs_t, src16, dst16, z128, d_in // 128, nb16)
    hs1 = _tc_conv(agg1, nd, ns, W1, b1, d_in // 128, W1.shape[1] // 128)
    agg2 = _sc_segsum_slices(hs1, src16, dst16, z128, W1.shape[1] // 128, nb16)
    t3 = _tc_conv_proj(agg2, nd, ns, W2, b2, Wcp, W2.shape[0] // 128)
    agg3p = _sc_segsum_esplit(t3, src32, dst32, z128, nb32)
    probs = _tc_softmax(agg3p, nd, bc)
    return probs[:N]


# speedup vs baseline: 2.9000x; 1.0143x over previous
import functools

import jax
import jax.numpy as jnp
from jax import lax
from jax.experimental import pallas as pl
from jax.experimental.pallas import tpu as pltpu
from jax.experimental.pallas import tpu_sc as plsc

_NC = 2
_NS = 16
_K = 128
_BK = 128
_CH = 40
_BM = 256

_N = 10000
_NPAD = 10240
_ROWS_PER_SUB = _NPAD // _NS


def _sc_mesh():
    return plsc.VectorSubcoreMesh(
        core_axis_name="c", subcore_axis_name="s", num_cores=_NC,
        num_subcores=_NS)


def _sc_degrees(src_e, dst_e, onesA_hbm, onesB_hbm, z128_hbm, nb):
    @functools.partial(
        pl.kernel,
        out_type=jax.ShapeDtypeStruct((_NC, _NPAD, 128), jnp.float32),
        mesh=_sc_mesh(),
        scratch_types=[
            pltpu.VMEM((nb, _BK), jnp.int32),
            pltpu.VMEM((nb, _BK), jnp.int32),
            pltpu.VMEM((_BK, 128), jnp.float32),
            pltpu.VMEM((_BK, 128), jnp.float32),
            pltpu.VMEM_SHARED((_NPAD, 128), jnp.float32),
        ],
    )
    def k(src_hbm, dst_hbm, onesA_in, onesB_in, z_in, out_hbm,
          srcv, dstv, onesA, onesB, acc):
        cid = lax.axis_index("c")
        sid = lax.axis_index("s")
        wid = cid * _NS + sid
        pltpu.sync_copy(src_hbm.at[wid], srcv)
        pltpu.sync_copy(dst_hbm.at[wid], dstv)
        pltpu.sync_copy(onesA_in, onesA)
        pltpu.sync_copy(onesB_in, onesB)
        r0 = sid * _ROWS_PER_SUB
        def zero(i, c):
            pltpu.sync_copy(z_in, acc.at[pl.ds(r0 + i * _K, _K)])
            return c
        lax.fori_loop(0, _ROWS_PER_SUB // _K, zero, 0)
        plsc.subcore_barrier()
        def scat(j, c):
            pltpu.sync_copy(onesA, acc.at[srcv.at[j]], add=True)
            pltpu.sync_copy(onesB, acc.at[dstv.at[j]], add=True)
            return c
        lax.fori_loop(0, nb, scat, 0)
        plsc.subcore_barrier()
        pltpu.sync_copy(acc.at[pl.ds(r0, _ROWS_PER_SUB)],
                        out_hbm.at[cid].at[pl.ds(r0, _ROWS_PER_SUB)])

    return k(src_e, dst_e, onesA_hbm, onesB_hbm, z128_hbm)


def _sc_segsum_slices(table, src_r, dst_r, z128_hbm, nsl, nb):
    nsl_sc = nsl // _NC
    nch = nb // _CH

    @functools.partial(
        pl.kernel,
        out_type=jax.ShapeDtypeStruct((nsl, _NPAD, 128), jnp.float32),
        mesh=_sc_mesh(),
        scratch_types=[
            pltpu.VMEM((_CH, _BK), jnp.int32),
            pltpu.VMEM((_CH, _BK), jnp.int32),
            pltpu.VMEM((_BK, 128), jnp.float32),
            pltpu.VMEM((_BK, 128), jnp.float32),
            pltpu.VMEM_SHARED((_NPAD, 128), jnp.float32),
            pltpu.SemaphoreType.DMA,
            pltpu.SemaphoreType.DMA,
        ],
    )
    def k(tab_hbm, src_hbm, dst_hbm, z_in, out_hbm,
          srcv, dstv, buf0, buf1, acc, sem0, sem1):
        cid = lax.axis_index("c")
        sid = lax.axis_index("s")
        bufs = (buf0, buf1)
        sems = (sem0, sem1)
        r0 = sid * _ROWS_PER_SUB
        for t in range(nsl_sc):
            sl = cid * nsl_sc + t
            pltpu.sync_copy(z_in, buf0)
            def zero(i, c):
                pltpu.sync_copy(buf0, acc.at[pl.ds(r0 + i * _K, _K)])
                return c
            lax.fori_loop(0, _ROWS_PER_SUB // _K, zero, 0)
            plsc.subcore_barrier()
            tab = tab_hbm.at[sl]
            for h in range(nch):
                pltpu.sync_copy(src_hbm.at[sid].at[pl.ds(h * _CH, _CH)], srcv)
                pltpu.sync_copy(dst_hbm.at[sid].at[pl.ds(h * _CH, _CH)], dstv)
                pltpu.async_copy(tab.at[srcv.at[0]], buf0, sem0)
                pltpu.async_copy(tab.at[srcv.at[1]], buf1, sem1)
                def outer(i0, c):
                    for b in range(2):
                        j = 2 * i0 + b
                        pltpu.make_async_copy(
                            tab.at[srcv.at[j]], bufs[b], sems[b]).wait()
                        pltpu.sync_copy(bufs[b], acc.at[dstv.at[j]], add=True)
                        @pl.when(j + 2 < _CH)
                        def _():
                            pltpu.async_copy(
                                tab.at[srcv.at[j + 2]], bufs[b], sems[b])
                    return c
                lax.fori_loop(0, _CH // 2, outer, 0)
            plsc.subcore_barrier()
            pltpu.sync_copy(acc.at[pl.ds(r0, _ROWS_PER_SUB)],
                            out_hbm.at[sl].at[pl.ds(r0, _ROWS_PER_SUB)])
            if t + 1 < nsl_sc:
                plsc.subcore_barrier()

    return k(table, src_r, dst_r, z128_hbm)


def _sc_segsum_esplit(table, src_e, dst_e, z128_hbm, nb):
    @functools.partial(
        pl.kernel,
        out_type=jax.ShapeDtypeStruct((_NC, _NPAD, 128), jnp.float32),
        mesh=_sc_mesh(),
        scratch_types=[
            pltpu.VMEM((nb, _BK), jnp.int32),
            pltpu.VMEM((nb, _BK), jnp.int32),
            pltpu.VMEM((_BK, 128), jnp.float32),
            pltpu.VMEM((_BK, 128), jnp.float32),
            pltpu.VMEM_SHARED((_NPAD, 128), jnp.float32),
            pltpu.SemaphoreType.DMA,
            pltpu.SemaphoreType.DMA,
        ],
    )
    def k(tab_hbm, src_hbm, dst_hbm, z_in, out_hbm,
          srcv, dstv, buf0, buf1, acc, sem0, sem1):
        cid = lax.axis_index("c")
        sid = lax.axis_index("s")
        wid = cid * _NS + sid
        pltpu.sync_copy(src_hbm.at[wid], srcv)
        pltpu.sync_copy(dst_hbm.at[wid], dstv)
        bufs = (buf0, buf1)
        sems = (sem0, sem1)
        r0 = sid * _ROWS_PER_SUB
        pltpu.sync_copy(z_in, buf0)
        def zero(i, c):
            pltpu.sync_copy(buf0, acc.at[pl.ds(r0 + i * _K, _K)])
            return c
        lax.fori_loop(0, _ROWS_PER_SUB // _K, zero, 0)
        plsc.subcore_barrier()
        pltpu.async_copy(tab_hbm.at[srcv.at[0]], buf0, sem0)
        pltpu.async_copy(tab_hbm.at[srcv.at[1]], buf1, sem1)
        def outer(i0, c):
            for b in range(2):
                j = 2 * i0 + b
                pltpu.make_async_copy(
                    tab_hbm.at[srcv.at[j]], bufs[b], sems[b]).wait()
                pltpu.sync_copy(bufs[b], acc.at[dstv.at[j]], add=True)
                @pl.when(j + 2 < nb)
                def _():
                    pltpu.async_copy(
                        tab_hbm.at[srcv.at[j + 2]], bufs[b], sems[b])
            return c
        lax.fori_loop(0, nb // 2, outer, 0)
        plsc.subcore_barrier()
        pltpu.sync_copy(acc.at[pl.ds(r0, _ROWS_PER_SUB)],
                        out_hbm.at[cid].at[pl.ds(r0, _ROWS_PER_SUB)])

    return k(table, src_e, dst_e, z128_hbm)


def _tc_norms_xs(xpad, degs):
    def body(x_ref, degs_ref, xs_ref, ns_ref, nd_ref):
        dsrc = degs_ref[0, :, 0] + degs_ref[1, :, 0]
        ddst = degs_ref[0, :, 1] + degs_ref[1, :, 1]
        ns = lax.rsqrt(jnp.maximum(dsrc, 1.0))
        nd = lax.rsqrt(jnp.maximum(ddst, 1.0))
        ns_ref[...] = ns[:, None]
        nd_ref[...] = nd[:, None]
        xs_ref[0] = x_ref[...] * ns[:, None]

    grid = (_NPAD // _BM, 2)
    return pl.pallas_call(
        body,
        grid=grid,
        in_specs=[
            pl.BlockSpec((_BM, 128), lambda i, j: (i, j)),
            pl.BlockSpec((_NC, _BM, 128), lambda i, j: (0, i, 0)),
        ],
        out_specs=[
            pl.BlockSpec((1, _BM, 128), lambda i, j: (j, i, 0)),
            pl.BlockSpec((_BM, 1), lambda i, j: (i, 0)),
            pl.BlockSpec((_BM, 1), lambda i, j: (i, 0)),
        ],
        out_shape=[
            jax.ShapeDtypeStruct((2, _NPAD, 128), jnp.float32),
            jax.ShapeDtypeStruct((_NPAD, 1), jnp.float32),
            jax.ShapeDtypeStruct((_NPAD, 1), jnp.float32),
        ],
    )(xpad, degs)


def _tc_conv(agg_t, nd, ns, W, b, nsl_in, nsl_out):
    K = nsl_in * 128
    b_r = b.reshape(nsl_out, 1, 128)

    def body(agg_ref, nd_ref, ns_ref, W_ref, b_ref, out_ref):
        z = jnp.zeros((_BM, 128), jnp.float32)
        for s in range(nsl_in):
            z = z + jnp.dot(agg_ref[s], W_ref[s * 128:(s + 1) * 128, :],
                            preferred_element_type=jnp.float32)
        z = z * nd_ref[...] + b_ref[0]
        out_ref[0] = jnp.maximum(z, 0.0) * ns_ref[...]

    grid = (_NPAD // _BM, nsl_out)
    return pl.pallas_call(
        body,
        grid=grid,
        in_specs=[
            pl.BlockSpec((nsl_in, _BM, 128), lambda i, j: (0, i, 0)),
            pl.BlockSpec((_BM, 1), lambda i, j: (i, 0)),
            pl.BlockSpec((_BM, 1), lambda i, j: (i, 0)),
            pl.BlockSpec((K, 128), lambda i, j: (0, j)),
            pl.BlockSpec((1, 1, 128), lambda i, j: (j, 0, 0)),
        ],
        out_specs=pl.BlockSpec((1, _BM, 128), lambda i, j: (j, i, 0)),
        out_shape=jax.ShapeDtypeStruct((nsl_out, _NPAD, 128), jnp.float32),
    )(agg_t, nd, ns, W, b_r)


def _tc_conv_proj(agg_t, nd, ns, W2, b2, Wcp, nsl_in):
    K = nsl_in * 128
    b_r = b2.reshape(1, K)

    def body(agg_ref, nd_ref, ns_ref, W_ref, b_ref, Wc_ref, out_ref):
        z = jnp.zeros((_BM, K), jnp.float32)
        for s in range(nsl_in):
            z = z + jnp.dot(agg_ref[s], W_ref[s * 128:(s + 1) * 128, :],
                            preferred_element_type=jnp.float32)
        z = jnp.maximum(z * nd_ref[...] + b_ref[...], 0.0) * ns_ref[...]
        out_ref[...] = jnp.dot(z, Wc_ref[...],
                               preferred_element_type=jnp.float32)

    grid = (_NPAD // _BM,)
    return pl.pallas_call(
        body,
        grid=grid,
        in_specs=[
            pl.BlockSpec((nsl_in, _BM, 128), lambda i: (0, i, 0)),
            pl.BlockSpec((_BM, 1), lambda i: (i, 0)),
            pl.BlockSpec((_BM, 1), lambda i: (i, 0)),
            pl.BlockSpec((K, K), lambda i: (0, 0)),
            pl.BlockSpec((1, K), lambda i: (0, 0)),
            pl.BlockSpec((K, 128), lambda i: (0, 0)),
        ],
        out_specs=pl.BlockSpec((_BM, 128), lambda i: (i, 0)),
        out_shape=jax.ShapeDtypeStruct((_NPAD, 128), jnp.float32),
    )(agg_t, nd, ns, W2, b_r, Wcp)


def _tc_softmax(agg3p, nd, bc):
    bc_r = bc.reshape(1, 64)

    def body(p_ref, nd_ref, bc_ref, out_ref):
        z = (p_ref[0, :, :64] + p_ref[1, :, :64]) * nd_ref[...] + bc_ref[...]
        m = jnp.max(z, axis=1, keepdims=True)
        e = jnp.exp(z - m)
        out_ref[...] = e / jnp.sum(e, axis=1, keepdims=True)

    grid = (_NPAD // _BM,)
    return pl.pallas_call(
        body,
        grid=grid,
        in_specs=[
            pl.BlockSpec((_NC, _BM, 128), lambda i: (0, i, 0)),
            pl.BlockSpec((_BM, 1), lambda i: (i, 0)),
            pl.BlockSpec((1, 64), lambda i: (0, 0)),
        ],
        out_specs=pl.BlockSpec((_BM, 64), lambda i: (i, 0)),
        out_shape=jax.ShapeDtypeStruct((_NPAD, 64), jnp.float32),
    )(agg3p, nd, bc_r)


def kernel(x, edge_index, W1, b1, W2, b2, Wc, bc):
    N = x.shape[0]
    E = edge_index.shape[1]
    d_in = x.shape[1]

    epad = -(-E // (_NC * _NS * _BK)) * (_NC * _NS * _BK)
    src = edge_index[0]
    dst = edge_index[1]
    pad = epad - E
    srcp = jnp.concatenate([src, jnp.full((pad,), N, jnp.int32)])
    dstp = jnp.concatenate([dst, jnp.full((pad,), N, jnp.int32)])
    src16 = srcp.reshape(_NS, epad // (_NS * _BK), _BK)
    dst16 = dstp.reshape(_NS, epad // (_NS * _BK), _BK)
    src32 = srcp.reshape(_NC * _NS, epad // (_NC * _NS * _BK), _BK)
    dst32 = dstp.reshape(_NC * _NS, epad // (_NC * _NS * _BK), _BK)
    nb16 = epad // (_NS * _BK)
    nb32 = epad // (_NC * _NS * _BK)

    xpad = jnp.pad(x, ((0, _NPAD - N), (0, 0)))
    z128 = jnp.zeros((_K, 128), jnp.float32)
    onesA = jnp.zeros((_BK, 128), jnp.float32).at[:, 0].set(1.0)
    onesB = jnp.zeros((_BK, 128), jnp.float32).at[:, 1].set(1.0)
    Wcp = jnp.pad(Wc, ((0, 0), (0, 128 - Wc.shape[1])))

    degs = _sc_degrees(src32, dst32, onesA, onesB, z128, nb32)
    xs_t, ns, nd = _tc_norms_xs(xpad, degs)
    agg1 = _sc_segsum_slices(xs_t, src16, dst16, z128, d_in // 128, nb16)
    hs1 = _tc_conv(agg1, nd, ns, W1, b1, d_in // 128, W1.shape[1] // 128)
    agg2 = _sc_segsum_slices(hs1, src16, dst16, z128, W1.shape[1] // 128, nb16)
    t3 = _tc_conv_proj(agg2, nd, ns, W2, b2, Wcp, W2.shape[0] // 128)
    agg3p = _sc_segsum_esplit(t3, src32, dst32, z128, nb32)
    probs = _tc_softmax(agg3p, nd, bc)
    return probs[:N]
